# baseline (speedup 1.0000x reference)
import functools
import jax
import jax.numpy as jnp
from jax import lax
from jax.experimental import pallas as pl
from jax.experimental.pallas import tpu as pltpu
from jax.experimental.pallas import tpu_sc as plsc

N_KEEP = 48 * 256
BATCH = 1024
NC, NS = 2, 16
NW = NC * NS
ROWS_PER_W = BATCH // NW
CHUNK = 4
NCHUNK = ROWS_PER_W // CHUNK
NBUF = 2

_mesh = plsc.VectorSubcoreMesh(core_axis_name="c", subcore_axis_name="s")


@functools.partial(
    pl.kernel,
    mesh=_mesh,
    out_type=jax.ShapeDtypeStruct((BATCH, N_KEEP), jnp.float32),
    scratch_types=[
        pltpu.VMEM((NBUF, CHUNK, N_KEEP), jnp.float32),
        pltpu.SemaphoreType.DMA,
        pltpu.SemaphoreType.DMA,
    ],
)
def _sc_copy(in_hbm, out_hbm, buf, sem_in, sem_out):
    wid = lax.axis_index("s") * NC + lax.axis_index("c")
    base = wid * ROWS_PER_W

    def in_copy(g, slot):
        r0 = base + g * CHUNK
        return pltpu.make_async_copy(
            in_hbm.at[pl.ds(r0, CHUNK), pl.ds(0, N_KEEP)], buf.at[slot], sem_in)

    def out_copy(g, slot):
        r0 = base + g * CHUNK
        return pltpu.make_async_copy(
            buf.at[slot], out_hbm.at[pl.ds(r0, CHUNK)], sem_out)

    in_copy(0, 0).start()
    in_copy(1, 1).start()
    for g in range(NCHUNK):
        slot = g % NBUF
        in_copy(g, slot).wait()
        oc = out_copy(g, slot)
        oc.start()
        if g + NBUF < NCHUNK:
            oc.wait()
            in_copy(g + NBUF, slot).start()
        else:
            oc.wait()


def kernel(inputs):
    return _sc_copy(inputs)
